# baseline (speedup 1.0000x reference)
import functools

import jax
import jax.numpy as jnp
from jax import lax
from jax.experimental import pallas as pl
from jax.experimental.pallas import tpu as pltpu
from jax.experimental.pallas import tpu_sc as plsc

F32 = jnp.float32
I32 = jnp.int32
K = 32
_NC, _NS = 2, 16
_NW = _NC * _NS
_INV_LBR = 1.0 / (1.0 + 1e-5) ** 0.5


def _pick_chunk(b_per_w, d):
    budget = 220 * 1024 // (4 * d)
    c = b_per_w
    while c > budget or c % 8 != 0:
        if c % 2 != 0:
            return 8
        c //= 2
    return max(c, 8)


@functools.lru_cache(maxsize=None)
def _sc_gather_fn(v_rows, d, b_total):
    b_per_w = b_total // _NW
    chunk = _pick_chunk(b_per_w, d)
    iters = b_per_w // chunk
    mesh = plsc.VectorSubcoreMesh(core_axis_name="c", subcore_axis_name="s")

    @functools.partial(
        pl.kernel,
        out_type=jax.ShapeDtypeStruct((b_total, d), F32),
        mesh=mesh,
        scratch_types=[
            pltpu.VMEM((chunk,), I32),
            pltpu.VMEM((chunk,), I32),
            pltpu.VMEM((chunk, d), F32),
            pltpu.VMEM((chunk, d), F32),
            pltpu.SemaphoreType.DMA,
            pltpu.SemaphoreType.DMA,
            pltpu.SemaphoreType.DMA,
            pltpu.SemaphoreType.DMA,
        ],
    )
    def gather_kernel(table_hbm, idx_hbm, out_hbm,
                      i0, i1, r0, r1, sg0, sg1, sw0, sw1):
        wid = lax.axis_index("s") * _NC + lax.axis_index("c")
        base0 = wid * b_per_w
        idxb, rowb, sg, sw = [i0, i1], [r0, r1], [sg0, sg1], [sw0, sw1]
        gh, wh = [None, None], [None, None]
        pltpu.sync_copy(idx_hbm.at[pl.ds(base0, chunk)], i0)
        gh[0] = pltpu.async_copy(table_hbm.at[i0], r0, sg0)
        for t in range(iters):
            b = t & 1
            nb = 1 - b
            if t + 1 < iters:
                pltpu.sync_copy(
                    idx_hbm.at[pl.ds(base0 + (t + 1) * chunk, chunk)],
                    idxb[nb])
                if wh[nb] is not None:
                    wh[nb].wait()
                gh[nb] = pltpu.async_copy(table_hbm.at[idxb[nb]],
                                          rowb[nb], sg[nb])
            gh[b].wait()
            wh[b] = pltpu.async_copy(
                rowb[b], out_hbm.at[pl.ds(base0 + t * chunk, chunk)], sw[b])
        for h in wh:
            if h is not None:
                h.wait()

    return gather_kernel


def _sc_gather(table, idx):
    v_rows, d = table.shape
    (b_total,) = idx.shape
    assert b_total % (8 * _NW) == 0 and d % 128 == 0, (table.shape, idx.shape)
    return _sc_gather_fn(v_rows, d, b_total)(table, idx)


def _matmul(a, b):
    return jnp.dot(a, b, preferred_element_type=F32)


def _lbr_call(x, p):
    r, ci = x.shape
    co = p['W'].shape[1]
    blk = min(r, 512)

    def body(x_ref, w_ref, b_ref, g_ref, be_ref, o_ref):
        h = _matmul(x_ref[...], w_ref[...]) + b_ref[...]
        h = h * _INV_LBR * g_ref[...] + be_ref[...]
        o_ref[...] = jnp.maximum(h, 0.0)

    return pl.pallas_call(
        body,
        grid=(r // blk,),
        in_specs=[
            pl.BlockSpec((blk, ci), lambda i: (i, 0)),
            pl.BlockSpec((ci, co), lambda i: (0, 0)),
            pl.BlockSpec((1, co), lambda i: (0, 0)),
            pl.BlockSpec((1, co), lambda i: (0, 0)),
            pl.BlockSpec((1, co), lambda i: (0, 0)),
        ],
        out_specs=pl.BlockSpec((blk, co), lambda i: (i, 0)),
        out_shape=jax.ShapeDtypeStruct((r, co), F32),
    )(x, p['W'], p['b'].reshape(1, co), p['g'].reshape(1, co),
      p['be'].reshape(1, co))


def _dense_relu_call(x, w, b):
    r, ci = x.shape
    co = w.shape[1]
    blk = min(r, 512)

    def body(x_ref, w_ref, b_ref, o_ref):
        o_ref[...] = jnp.maximum(_matmul(x_ref[...], w_ref[...]) + b_ref[...],
                                 0.0)

    return pl.pallas_call(
        body,
        grid=(r // blk,),
        in_specs=[
            pl.BlockSpec((blk, ci), lambda i: (i, 0)),
            pl.BlockSpec((ci, co), lambda i: (0, 0)),
            pl.BlockSpec((1, co), lambda i: (0, 0)),
        ],
        out_specs=pl.BlockSpec((blk, co), lambda i: (i, 0)),
        out_shape=jax.ShapeDtypeStruct((r, co), F32),
    )(x, w, b.reshape(1, co))


def _node_transform_call(x, p):
    r, c = x.shape
    blk = min(r, 256)

    def body(x_ref, wi_ref, bi_ref, w_ref, ws_ref, wd_ref,
             va_ref, ad_ref):
        x1 = jnp.maximum(_matmul(x_ref[...], wi_ref[...]) + bi_ref[...], 0.0)
        va_ref[:, 0:c] = _matmul(x1, w_ref[...])
        va_ref[:, c:2 * c] = _matmul(x1, ws_ref[...])
        ad_ref[...] = _matmul(x1, wd_ref[...])

    outs = pl.pallas_call(
        body,
        grid=(r // blk,),
        in_specs=[
            pl.BlockSpec((blk, c), lambda i: (i, 0)),
            pl.BlockSpec((c, c), lambda i: (0, 0)),
            pl.BlockSpec((1, c), lambda i: (0, 0)),
            pl.BlockSpec((c, c), lambda i: (0, 0)),
            pl.BlockSpec((c, c), lambda i: (0, 0)),
            pl.BlockSpec((c, c), lambda i: (0, 0)),
        ],
        out_specs=[pl.BlockSpec((blk, 2 * c), lambda i: (i, 0)),
                   pl.BlockSpec((blk, c), lambda i: (i, 0))],
        out_shape=[jax.ShapeDtypeStruct((r, 2 * c), F32),
                   jax.ShapeDtypeStruct((r, c), F32)],
    )(x, p['lin_in_W'], p['lin_in_b'].reshape(1, c),
      p['W'], p['W_src'], p['W_dst'])
    return outs


def _edge_attn_call(gva, pj, a_dst, pos, p, n, c):
    blk = max(2048 // c * 8, 8)
    blk = min(blk, n)
    while n % blk:
        blk //= 2

    def body(gva_ref, gp_ref, ad_ref, pp_ref,
             pw1_ref, pb1_ref, pw2_ref, pb2_ref,
             aw1_ref, ab1_ref, aw2_ref, ab2_ref,
             lo_ref, lb_ref, o_ref):
        kb = K * blk
        gx3 = gva_ref[0][:, :, 0:c]
        ga3 = gva_ref[0][:, :, c:2 * c]
        gp3 = gp_ref[0]
        ad = ad_ref[0]
        pd = pp_ref[0]

        rel3 = jnp.broadcast_to(pd[None], (K, blk, 3)) - gp3
        rel = rel3.reshape(kb, 3)
        h = (rel[:, 0:1] * pw1_ref[0:1, :] + rel[:, 1:2] * pw1_ref[1:2, :]
             + rel[:, 2:3] * pw1_ref[2:3, :]) + pb1_ref[...]
        h = jnp.maximum(h, 0.0)
        delta = jnp.maximum(_matmul(h, pw2_ref[...]) + pb2_ref[...], 0.0)
        delta3 = delta.reshape(K, blk, c)

        alpha0 = (jnp.broadcast_to(ad[None], (K, blk, c)) - ga3
                  + delta3).reshape(kb, c)
        t = jnp.maximum(_matmul(alpha0, aw1_ref[...]) + ab1_ref[...], 0.0)
        alpha = jnp.maximum(_matmul(t, aw2_ref[...]) + ab2_ref[...], 0.0)
        alpha3 = alpha.reshape(K, blk, c)

        m = alpha3[0]
        for k in range(1, K):
            m = jnp.maximum(m, alpha3[k])
        e3 = jnp.exp(alpha3 - m[None])
        s = e3[0]
        for k in range(1, K):
            s = s + e3[k]
        w3 = e3 / s[None]
        acc = w3[0] * (gx3[0] + delta3[0])
        for k in range(1, K):
            acc = acc + w3[k] * (gx3[k] + delta3[k])

        o_ref[0] = jnp.maximum(_matmul(acc, lo_ref[...]) + lb_ref[...], 0.0)

    return pl.pallas_call(
        body,
        grid=(2, n // blk),
        in_specs=[
            pl.BlockSpec((1, K, blk, 2 * c), lambda b, i: (b, 0, i, 0)),
            pl.BlockSpec((1, K, blk, 3), lambda b, i: (b, 0, i, 0)),
            pl.BlockSpec((1, blk, c), lambda b, i: (b, i, 0)),
            pl.BlockSpec((1, blk, 3), lambda b, i: (b, i, 0)),
            pl.BlockSpec((3, 64), lambda b, i: (0, 0)),
            pl.BlockSpec((1, 64), lambda b, i: (0, 0)),
            pl.BlockSpec((64, c), lambda b, i: (0, 0)),
            pl.BlockSpec((1, c), lambda b, i: (0, 0)),
            pl.BlockSpec((c, 64), lambda b, i: (0, 0)),
            pl.BlockSpec((1, 64), lambda b, i: (0, 0)),
            pl.BlockSpec((64, c), lambda b, i: (0, 0)),
            pl.BlockSpec((1, c), lambda b, i: (0, 0)),
            pl.BlockSpec((c, c), lambda b, i: (0, 0)),
            pl.BlockSpec((1, c), lambda b, i: (0, 0)),
        ],
        out_specs=pl.BlockSpec((1, blk, c), lambda b, i: (b, i, 0)),
        out_shape=jax.ShapeDtypeStruct((2, n, c), F32),
    )(gva, pj, a_dst, pos,
      p['pos_W1'], p['pos_b1'].reshape(1, 64), p['pos_W2'],
      p['pos_b2'].reshape(1, c),
      p['attn_W1'], p['attn_b1'].reshape(1, 64), p['attn_W2'],
      p['attn_b2'].reshape(1, c),
      p['lin_out_W'], p['lin_out_b'].reshape(1, c))


def _transformer_block(x_flat, pos, pj, idx, n, c, p):
    va, a_dst = _node_transform_call(x_flat, p)
    gva = _sc_gather(va, idx).reshape(2, K, n, 2 * c)
    out = _edge_attn_call(gva, pj, a_dst.reshape(2, n, c), pos, p, n, c)
    return out.reshape(2 * n, c)


def _kmajor_idx(nbr, n):
    offs = (jnp.arange(2, dtype=I32) * n)[:, None, None]
    return jnp.transpose(nbr + offs, (0, 2, 1)).reshape(-1)


def _knn_call(query, base_t, kk, exclude_self, want_pj=False):
    _, q, _ = query.shape
    nb = base_t.shape[2]
    bq = min(q, 256)
    inf = float('inf')
    big = 2 ** 30

    def body(q_ref, b_ref, o_ref, *pj_ref):
        qs = q_ref[0]
        bx = b_ref[0, 0:1, :]
        by = b_ref[0, 1:2, :]
        bz = b_ref[0, 2:3, :]
        dx = qs[:, 0:1] - bx
        dy = qs[:, 1:2] - by
        dz = qs[:, 2:3] - bz
        d = dx * dx + dy * dy + dz * dz
        col = lax.broadcasted_iota(I32, (bq, nb), 1)
        if exclude_self:
            row = (lax.broadcasted_iota(I32, (bq, nb), 0)
                   + pl.program_id(1) * bq)
            d = jnp.where(row == col, inf, d)
        cols = []
        for t in range(kk):
            mval = jnp.min(d, axis=1, keepdims=True)
            sel = jnp.where(d == mval, col, big)
            midx = jnp.min(sel, axis=1, keepdims=True)
            cols.append(midx)
            m2 = col == midx
            if want_pj:
                pj = pj_ref[0]
                pj[0, t, :, 0:1] = jnp.min(
                    jnp.where(m2, bx, inf), axis=1, keepdims=True)
                pj[0, t, :, 1:2] = jnp.min(
                    jnp.where(m2, by, inf), axis=1, keepdims=True)
                pj[0, t, :, 2:3] = jnp.min(
                    jnp.where(m2, bz, inf), axis=1, keepdims=True)
            d = jnp.where(m2, inf, d)
        o_ref[0] = jnp.concatenate(cols, axis=1)

    out_specs = [pl.BlockSpec((1, bq, kk), lambda b, i: (b, i, 0))]
    out_shape = [jax.ShapeDtypeStruct((2, q, kk), I32)]
    if want_pj:
        out_specs.append(pl.BlockSpec((1, kk, bq, 3), lambda b, i: (b, 0, i, 0)))
        out_shape.append(jax.ShapeDtypeStruct((2, kk, q, 3), F32))
    res = pl.pallas_call(
        body,
        grid=(2, q // bq),
        in_specs=[
            pl.BlockSpec((1, bq, 3), lambda b, i: (b, i, 0)),
            pl.BlockSpec((1, 3, nb), lambda b, i: (b, 0, 0)),
        ],
        out_specs=out_specs,
        out_shape=out_shape,
    )(query, base_t)
    return res if want_pj else res[0]


def _fps_call(posr, m):
    n8 = posr.shape[3]
    big = 2 ** 30
    neg = -1e30

    def body(p_ref, oi_ref, op_ref):
        px = p_ref[0, 0]
        py = p_ref[0, 1]
        pz = p_ref[0, 2]
        fiota = (lax.broadcasted_iota(I32, (8, n8), 0) * n8
                 + lax.broadcasted_iota(I32, (8, n8), 1))
        oi_ref[0, 0:1, 0:1] = jnp.zeros((1, 1), I32)
        op_ref[0, 0:1, 0:1] = p_ref[0, 0, 0, 0].reshape(1, 1)
        op_ref[0, 0:1, 1:2] = p_ref[0, 1, 0, 0].reshape(1, 1)
        op_ref[0, 0:1, 2:3] = p_ref[0, 2, 0, 0].reshape(1, 1)

        def step(i, carry):
            dist, lx, ly, lz = carry
            dxx = px - lx
            dyy = py - ly
            dzz = pz - lz
            d = dxx * dxx + dyy * dyy + dzz * dzz
            dist = jnp.minimum(dist, d)
            mx = jnp.max(dist)
            sel = jnp.where(dist == mx, fiota, big)
            nxt = jnp.min(sel)
            mask = fiota == nxt
            nlx = jnp.max(jnp.where(mask, px, neg))
            nly = jnp.max(jnp.where(mask, py, neg))
            nlz = jnp.max(jnp.where(mask, pz, neg))
            oi_ref[0, pl.ds(i, 1), 0:1] = nxt.reshape(1, 1)
            op_ref[0, pl.ds(i, 1), 0:1] = nlx.reshape(1, 1)
            op_ref[0, pl.ds(i, 1), 1:2] = nly.reshape(1, 1)
            op_ref[0, pl.ds(i, 1), 2:3] = nlz.reshape(1, 1)
            return dist, nlx, nly, nlz

        init = (jnp.full((8, n8), jnp.inf, F32),
                p_ref[0, 0, 0, 0], p_ref[0, 1, 0, 0], p_ref[0, 2, 0, 0])
        lax.fori_loop(1, m, step, init)

    return pl.pallas_call(
        body,
        grid=(2,),
        in_specs=[pl.BlockSpec((1, 3, 8, n8), lambda b: (b, 0, 0, 0))],
        out_specs=[
            pl.BlockSpec((1, m, 1), lambda b: (b, 0, 0)),
            pl.BlockSpec((1, m, 3), lambda b: (b, 0, 0)),
        ],
        out_shape=[
            jax.ShapeDtypeStruct((2, m, 1), I32),
            jax.ShapeDtypeStruct((2, m, 3), F32),
        ],
    )(posr)


def _down_max_call(gh, m, c):
    blk = min(m, 256)

    def body(g_ref, o_ref):
        g3 = g_ref[0]
        acc = g3[0]
        for k in range(1, K):
            acc = jnp.maximum(acc, g3[k])
        o_ref[0] = acc

    return pl.pallas_call(
        body,
        grid=(2, m // blk),
        in_specs=[pl.BlockSpec((1, K, blk, c), lambda b, i: (b, 0, i, 0))],
        out_specs=pl.BlockSpec((1, blk, c), lambda b, i: (b, i, 0)),
        out_shape=jax.ShapeDtypeStruct((2, m, c), F32),
    )(gh)


def _up_interp_call(gx, pj, pos, lbrx, n, c):
    blk = min(n, 512)

    def body(gx_ref, pj_ref, pp_ref, lx_ref, o_ref):
        pd = pp_ref[0]
        ws = None
        acc = None
        for k in range(3):
            pj = pj_ref[0, k]
            dd = pd - pj
            d = (dd[:, 0:1] * dd[:, 0:1] + dd[:, 1:2] * dd[:, 1:2]
                 + dd[:, 2:3] * dd[:, 2:3])
            w = 1.0 / jnp.maximum(d, 1e-16)
            term = gx_ref[0, k] * w
            ws = w if ws is None else ws + w
            acc = term if acc is None else acc + term
        o_ref[0] = lx_ref[0] + acc / ws

    return pl.pallas_call(
        body,
        grid=(2, n // blk),
        in_specs=[
            pl.BlockSpec((1, 3, blk, c), lambda b, i: (b, 0, i, 0)),
            pl.BlockSpec((1, 3, blk, 3), lambda b, i: (b, 0, i, 0)),
            pl.BlockSpec((1, blk, 3), lambda b, i: (b, i, 0)),
            pl.BlockSpec((1, blk, c), lambda b, i: (b, i, 0)),
        ],
        out_specs=pl.BlockSpec((1, blk, c), lambda b, i: (b, i, 0)),
        out_shape=jax.ShapeDtypeStruct((2, n, c), F32),
    )(gx, pj, pos, lbrx)


def _head_call(x, hp):
    r = x.shape[0]
    blk = min(r, 512)

    def ln(h, g, b):
        mu = jnp.mean(h, axis=-1, keepdims=True)
        var = jnp.mean((h - mu) ** 2, axis=-1, keepdims=True)
        return (h - mu) / jnp.sqrt(var + 1e-5) * g + b

    def body(x_ref, c1w, c1b, c2w, c2b, c3w, c3b, g1, b1, g2, b2, o_ref):
        h = _matmul(x_ref[...], c1w[...]) + c1b[...]
        h = ln(h, g1[...], b1[...])
        h = _matmul(h, c2w[...]) + c2b[...]
        h = ln(h, g2[...], b2[...])
        h = _matmul(h, c3w[...]) + c3b[...]
        sig = 1.0 / (1.0 + jnp.exp(-h))
        o_ref[...] = sig * 2.0 - 1.0

    return pl.pallas_call(
        body,
        grid=(r // blk,),
        in_specs=[
            pl.BlockSpec((blk, 128), lambda i: (i, 0)),
            pl.BlockSpec((128, 32), lambda i: (0, 0)),
            pl.BlockSpec((1, 32), lambda i: (0, 0)),
            pl.BlockSpec((32, 32), lambda i: (0, 0)),
            pl.BlockSpec((1, 32), lambda i: (0, 0)),
            pl.BlockSpec((32, 3), lambda i: (0, 0)),
            pl.BlockSpec((1, 3), lambda i: (0, 0)),
            pl.BlockSpec((1, 32), lambda i: (0, 0)),
            pl.BlockSpec((1, 32), lambda i: (0, 0)),
            pl.BlockSpec((1, 32), lambda i: (0, 0)),
            pl.BlockSpec((1, 32), lambda i: (0, 0)),
        ],
        out_specs=pl.BlockSpec((blk, 3), lambda i: (i, 0)),
        out_shape=jax.ShapeDtypeStruct((r, 3), F32),
    )(x, hp['c1W'], hp['c1b'].reshape(1, 32), hp['c2W'],
      hp['c2b'].reshape(1, 32), hp['c3W'], hp['c3b'].reshape(1, 3),
      hp['ln1_g'].reshape(1, 32), hp['ln1_b'].reshape(1, 32),
      hp['ln2_g'].reshape(1, 32), hp['ln2_b'].reshape(1, 32))


def _pos_r(pos, n):
    return jnp.transpose(pos, (0, 2, 1)).reshape(2, 3, 8, n // 8)


def _transition_down(x_flat, pos, n, m, p):
    idc, sub_pos = _fps_call(_pos_r(pos, n), m)
    base_t = jnp.transpose(pos, (0, 2, 1))
    nbr = _knn_call(sub_pos, base_t, K, False)
    h = _lbr_call(x_flat, p)
    idx = _kmajor_idx(nbr, n)
    gh = _sc_gather(h, idx).reshape(2, K, m, h.shape[1])
    out = _down_max_call(gh, m, h.shape[1])
    return out.reshape(2 * m, h.shape[1]), sub_pos


def _transition_up(x_flat, xsub_flat, pos, pos_sub, n, m, p_sub, p_mlp):
    xs = _lbr_call(xsub_flat, p_sub)
    c = xs.shape[1]
    sub_t = jnp.transpose(pos_sub, (0, 2, 1))
    nbr, pj = _knn_call(pos, sub_t, 3, False, want_pj=True)
    offs = (jnp.arange(2, dtype=I32) * m)[:, None, None]
    idx = jnp.transpose(nbr + offs, (0, 2, 1)).reshape(-1)
    gx = _sc_gather(xs, idx).reshape(2, 3, n, c)
    lbrx = _lbr_call(x_flat, p_mlp)
    out = _up_interp_call(gx, pj, pos, lbrx.reshape(2, n, c), n, c)
    return out.reshape(2 * n, c)


def kernel(cloud, params):
    p = params
    n0 = cloud.shape[1]
    n1, n2 = n0 // 4, n0 // 16
    pos0 = cloud
    pos0_flat = pos0.reshape(2 * n0, 3)
    pos0_t = jnp.transpose(pos0, (0, 2, 1))

    x = _lbr_call(pos0_flat, p['mlp_in'])
    nbr0, pj0 = _knn_call(pos0, pos0_t, K, True, want_pj=True)
    idx0 = _kmajor_idx(nbr0, n0)
    x0 = _transformer_block(x, pos0, pj0, idx0, n0, 128, p['t_in'])

    x1, pos1 = _transition_down(x0, pos0, n0, n1, p['td0'])
    pos1_t = jnp.transpose(pos1, (0, 2, 1))
    nbr1, pj1 = _knn_call(pos1, pos1_t, K, True, want_pj=True)
    idx1 = _kmajor_idx(nbr1, n1)
    x1 = _transformer_block(x1, pos1, pj1, idx1, n1, 256, p['t_d0'])

    x2, pos2 = _transition_down(x1, pos1, n1, n2, p['td1'])
    pos2_t = jnp.transpose(pos2, (0, 2, 1))
    nbr2, pj2 = _knn_call(pos2, pos2_t, K, True, want_pj=True)
    idx2 = _kmajor_idx(nbr2, n2)
    x2 = _transformer_block(x2, pos2, pj2, idx2, n2, 512, p['t_d1'])

    x2 = _dense_relu_call(x2, p['summit']['W'], p['summit']['b'])
    x2 = _transformer_block(x2, pos2, pj2, idx2, n2, 512, p['t_sum'])

    xu1 = _transition_up(x1, x2, pos1, pos2, n1, n2,
                         p['tu1_sub'], p['tu1_mlp'])
    xu1 = _transformer_block(xu1, pos1, pj1, idx1, n1, 256, p['t_u1'])

    xu0 = _transition_up(x0, xu1, pos0, pos1, n0, n1,
                         p['tu0_sub'], p['tu0_mlp'])
    xu0 = _transformer_block(xu0, pos0, pj0, idx0, n0, 128, p['t_u0'])

    out = _head_call(xu0, p['head'])
    return out.reshape(2, n0, 3)


# speedup vs baseline: 1.2450x; 1.2450x over previous
import functools

import jax
import jax.numpy as jnp
from jax import lax
from jax.experimental import pallas as pl
from jax.experimental.pallas import tpu as pltpu
from jax.experimental.pallas import tpu_sc as plsc

F32 = jnp.float32
I32 = jnp.int32
K = 32
_NC, _NS = 2, 16
_NW = _NC * _NS
_INV_LBR = 1.0 / (1.0 + 1e-5) ** 0.5


def _pick_chunk(b_per_w, d):
    budget = 360 * 1024 // (4 * d)
    c = b_per_w
    while c > budget or c % 8 != 0:
        if c % 2 != 0:
            return 8
        c //= 2
    return max(c, 8)


@functools.lru_cache(maxsize=None)
def _sc_gather_fn(v_rows, d, b_total):
    b_per_w = b_total // _NW
    chunk = _pick_chunk(b_per_w, d)
    iters = b_per_w // chunk
    mesh = plsc.VectorSubcoreMesh(core_axis_name="c", subcore_axis_name="s")

    @functools.partial(
        pl.kernel,
        out_type=jax.ShapeDtypeStruct((b_total, d), F32),
        mesh=mesh,
        scratch_types=[
            pltpu.VMEM((chunk,), I32),
            pltpu.VMEM((chunk, d), F32),
            pltpu.SemaphoreType.DMA,
        ],
    )
    def gather_kernel(table_hbm, idx_hbm, out_hbm, idx_v, rows_v, sem):
        wid = lax.axis_index("s") * _NC + lax.axis_index("c")
        base0 = wid * b_per_w
        for t in range(iters):
            base = base0 + t * chunk
            pltpu.sync_copy(idx_hbm.at[pl.ds(base, chunk)], idx_v)
            pltpu.async_copy(table_hbm.at[idx_v], rows_v, sem).wait()
            pltpu.sync_copy(rows_v, out_hbm.at[pl.ds(base, chunk)])

    return gather_kernel


def _sc_gather(table, idx):
    v_rows, d = table.shape
    (b_total,) = idx.shape
    assert b_total % (8 * _NW) == 0 and d % 128 == 0, (table.shape, idx.shape)
    return _sc_gather_fn(v_rows, d, b_total)(table, idx)


def _matmul(a, b):
    return jnp.dot(a, b, preferred_element_type=F32)


def _lbr_call(x, p):
    r, ci = x.shape
    co = p['W'].shape[1]
    blk = min(r, 512)

    def body(x_ref, w_ref, b_ref, g_ref, be_ref, o_ref):
        h = _matmul(x_ref[...], w_ref[...]) + b_ref[...]
        h = h * _INV_LBR * g_ref[...] + be_ref[...]
        o_ref[...] = jnp.maximum(h, 0.0)

    return pl.pallas_call(
        body,
        grid=(r // blk,),
        in_specs=[
            pl.BlockSpec((blk, ci), lambda i: (i, 0)),
            pl.BlockSpec((ci, co), lambda i: (0, 0)),
            pl.BlockSpec((1, co), lambda i: (0, 0)),
            pl.BlockSpec((1, co), lambda i: (0, 0)),
            pl.BlockSpec((1, co), lambda i: (0, 0)),
        ],
        out_specs=pl.BlockSpec((blk, co), lambda i: (i, 0)),
        out_shape=jax.ShapeDtypeStruct((r, co), F32),
    )(x, p['W'], p['b'].reshape(1, co), p['g'].reshape(1, co),
      p['be'].reshape(1, co))


def _dense_relu_call(x, w, b):
    r, ci = x.shape
    co = w.shape[1]
    blk = min(r, 512)

    def body(x_ref, w_ref, b_ref, o_ref):
        o_ref[...] = jnp.maximum(_matmul(x_ref[...], w_ref[...]) + b_ref[...],
                                 0.0)

    return pl.pallas_call(
        body,
        grid=(r // blk,),
        in_specs=[
            pl.BlockSpec((blk, ci), lambda i: (i, 0)),
            pl.BlockSpec((ci, co), lambda i: (0, 0)),
            pl.BlockSpec((1, co), lambda i: (0, 0)),
        ],
        out_specs=pl.BlockSpec((blk, co), lambda i: (i, 0)),
        out_shape=jax.ShapeDtypeStruct((r, co), F32),
    )(x, w, b.reshape(1, co))


def _node_transform_call(x, p):
    r, c = x.shape
    blk = min(r, 256)

    def body(x_ref, wi_ref, bi_ref, w_ref, ws_ref, wd_ref,
             va_ref, ad_ref):
        x1 = jnp.maximum(_matmul(x_ref[...], wi_ref[...]) + bi_ref[...], 0.0)
        va_ref[:, 0:c] = _matmul(x1, w_ref[...])
        va_ref[:, c:2 * c] = _matmul(x1, ws_ref[...])
        ad_ref[...] = _matmul(x1, wd_ref[...])

    outs = pl.pallas_call(
        body,
        grid=(r // blk,),
        in_specs=[
            pl.BlockSpec((blk, c), lambda i: (i, 0)),
            pl.BlockSpec((c, c), lambda i: (0, 0)),
            pl.BlockSpec((1, c), lambda i: (0, 0)),
            pl.BlockSpec((c, c), lambda i: (0, 0)),
            pl.BlockSpec((c, c), lambda i: (0, 0)),
            pl.BlockSpec((c, c), lambda i: (0, 0)),
        ],
        out_specs=[pl.BlockSpec((blk, 2 * c), lambda i: (i, 0)),
                   pl.BlockSpec((blk, c), lambda i: (i, 0))],
        out_shape=[jax.ShapeDtypeStruct((r, 2 * c), F32),
                   jax.ShapeDtypeStruct((r, c), F32)],
    )(x, p['lin_in_W'], p['lin_in_b'].reshape(1, c),
      p['W'], p['W_src'], p['W_dst'])
    return outs


def _edge_attn_call(gva, gp, a_dst, pos, p, n, c):
    blk = max(2048 // c * 8, 8)
    blk = min(blk, n)
    while n % blk:
        blk //= 2

    def body(gva_ref, gp_ref, ad_ref, pp_ref,
             pw1_ref, pb1_ref, pw2_ref, pb2_ref,
             aw1_ref, ab1_ref, aw2_ref, ab2_ref,
             lo_ref, lb_ref, o_ref):
        kb = K * blk
        gx3 = gva_ref[0][:, :, 0:c]
        ga3 = gva_ref[0][:, :, c:2 * c]
        gp3 = gp_ref[0][:, :, 0:3]
        ad = ad_ref[0]
        pd = pp_ref[0]

        rel3 = jnp.broadcast_to(pd[None], (K, blk, 3)) - gp3
        rel = rel3.reshape(kb, 3)
        h = (rel[:, 0:1] * pw1_ref[0:1, :] + rel[:, 1:2] * pw1_ref[1:2, :]
             + rel[:, 2:3] * pw1_ref[2:3, :]) + pb1_ref[...]
        h = jnp.maximum(h, 0.0)
        delta = jnp.maximum(_matmul(h, pw2_ref[...]) + pb2_ref[...], 0.0)
        delta3 = delta.reshape(K, blk, c)

        alpha0 = (jnp.broadcast_to(ad[None], (K, blk, c)) - ga3
                  + delta3).reshape(kb, c)
        t = jnp.maximum(_matmul(alpha0, aw1_ref[...]) + ab1_ref[...], 0.0)
        alpha = jnp.maximum(_matmul(t, aw2_ref[...]) + ab2_ref[...], 0.0)
        alpha3 = alpha.reshape(K, blk, c)

        m = alpha3[0]
        for k in range(1, K):
            m = jnp.maximum(m, alpha3[k])
        e3 = jnp.exp(alpha3 - m[None])
        s = e3[0]
        for k in range(1, K):
            s = s + e3[k]
        w3 = e3 / s[None]
        acc = w3[0] * (gx3[0] + delta3[0])
        for k in range(1, K):
            acc = acc + w3[k] * (gx3[k] + delta3[k])

        o_ref[0] = jnp.maximum(_matmul(acc, lo_ref[...]) + lb_ref[...], 0.0)

    return pl.pallas_call(
        body,
        grid=(2, n // blk),
        in_specs=[
            pl.BlockSpec((1, K, blk, 2 * c), lambda b, i: (b, 0, i, 0)),
            pl.BlockSpec((1, K, blk, 128), lambda b, i: (b, 0, i, 0)),
            pl.BlockSpec((1, blk, c), lambda b, i: (b, i, 0)),
            pl.BlockSpec((1, blk, 3), lambda b, i: (b, i, 0)),
            pl.BlockSpec((3, 64), lambda b, i: (0, 0)),
            pl.BlockSpec((1, 64), lambda b, i: (0, 0)),
            pl.BlockSpec((64, c), lambda b, i: (0, 0)),
            pl.BlockSpec((1, c), lambda b, i: (0, 0)),
            pl.BlockSpec((c, 64), lambda b, i: (0, 0)),
            pl.BlockSpec((1, 64), lambda b, i: (0, 0)),
            pl.BlockSpec((64, c), lambda b, i: (0, 0)),
            pl.BlockSpec((1, c), lambda b, i: (0, 0)),
            pl.BlockSpec((c, c), lambda b, i: (0, 0)),
            pl.BlockSpec((1, c), lambda b, i: (0, 0)),
        ],
        out_specs=pl.BlockSpec((1, blk, c), lambda b, i: (b, i, 0)),
        out_shape=jax.ShapeDtypeStruct((2, n, c), F32),
    )(gva, gp, a_dst, pos,
      p['pos_W1'], p['pos_b1'].reshape(1, 64), p['pos_W2'],
      p['pos_b2'].reshape(1, c),
      p['attn_W1'], p['attn_b1'].reshape(1, 64), p['attn_W2'],
      p['attn_b2'].reshape(1, c),
      p['lin_out_W'], p['lin_out_b'].reshape(1, c))


def _transformer_block(x_flat, pos, gp, idx, n, c, p):
    va, a_dst = _node_transform_call(x_flat, p)
    gva = _sc_gather(va, idx).reshape(2, K, n, 2 * c)
    out = _edge_attn_call(gva, gp, a_dst.reshape(2, n, c), pos, p, n, c)
    return out.reshape(2 * n, c)


def _kmajor_idx(nbr, n):
    offs = (jnp.arange(2, dtype=I32) * n)[:, None, None]
    return jnp.transpose(nbr + offs, (0, 2, 1)).reshape(-1)


def _knn_call(query, base_t, kk, exclude_self, want_pj=False):
    _, q, _ = query.shape
    nb = base_t.shape[2]
    bq = min(q, 256)
    inf = float('inf')
    big = 2 ** 30

    def body(q_ref, b_ref, o_ref, *pj_ref):
        qs = q_ref[0]
        bx = b_ref[0, 0:1, :]
        by = b_ref[0, 1:2, :]
        bz = b_ref[0, 2:3, :]
        dx = qs[:, 0:1] - bx
        dy = qs[:, 1:2] - by
        dz = qs[:, 2:3] - bz
        d = dx * dx + dy * dy + dz * dz
        col = lax.broadcasted_iota(I32, (bq, nb), 1)
        if exclude_self:
            row = (lax.broadcasted_iota(I32, (bq, nb), 0)
                   + pl.program_id(1) * bq)
            d = jnp.where(row == col, inf, d)
        cols = []
        for t in range(kk):
            mval = jnp.min(d, axis=1, keepdims=True)
            sel = jnp.where(d == mval, col, big)
            midx = jnp.min(sel, axis=1, keepdims=True)
            cols.append(midx)
            m2 = col == midx
            if want_pj:
                pj = pj_ref[0]
                pj[0, t, :, 0:1] = jnp.min(
                    jnp.where(m2, bx, inf), axis=1, keepdims=True)
                pj[0, t, :, 1:2] = jnp.min(
                    jnp.where(m2, by, inf), axis=1, keepdims=True)
                pj[0, t, :, 2:3] = jnp.min(
                    jnp.where(m2, bz, inf), axis=1, keepdims=True)
            d = jnp.where(m2, inf, d)
        o_ref[0] = jnp.concatenate(cols, axis=1)

    out_specs = [pl.BlockSpec((1, bq, kk), lambda b, i: (b, i, 0))]
    out_shape = [jax.ShapeDtypeStruct((2, q, kk), I32)]
    if want_pj:
        out_specs.append(pl.BlockSpec((1, kk, bq, 3), lambda b, i: (b, 0, i, 0)))
        out_shape.append(jax.ShapeDtypeStruct((2, kk, q, 3), F32))
    res = pl.pallas_call(
        body,
        grid=(2, q // bq),
        in_specs=[
            pl.BlockSpec((1, bq, 3), lambda b, i: (b, i, 0)),
            pl.BlockSpec((1, 3, nb), lambda b, i: (b, 0, 0)),
        ],
        out_specs=out_specs,
        out_shape=out_shape,
    )(query, base_t)
    return res if want_pj else res[0]


def _fps_call(posr, m):
    n8 = posr.shape[3]
    big = 2 ** 30
    neg = -1e30

    def body(p_ref, oi_ref, op_ref):
        px = p_ref[0, 0]
        py = p_ref[0, 1]
        pz = p_ref[0, 2]
        fiota = (lax.broadcasted_iota(I32, (8, n8), 0) * n8
                 + lax.broadcasted_iota(I32, (8, n8), 1))
        oi_ref[0, 0:1, 0:1] = jnp.zeros((1, 1), I32)
        op_ref[0, 0:1, 0:1] = p_ref[0, 0, 0, 0].reshape(1, 1)
        op_ref[0, 0:1, 1:2] = p_ref[0, 1, 0, 0].reshape(1, 1)
        op_ref[0, 0:1, 2:3] = p_ref[0, 2, 0, 0].reshape(1, 1)

        def step(i, carry):
            dist, lx, ly, lz = carry
            dxx = px - lx
            dyy = py - ly
            dzz = pz - lz
            d = dxx * dxx + dyy * dyy + dzz * dzz
            dist = jnp.minimum(dist, d)
            mx = jnp.max(dist)
            sel = jnp.where(dist == mx, fiota, big)
            nxt = jnp.min(sel)
            mask = fiota == nxt
            nlx = jnp.max(jnp.where(mask, px, neg))
            nly = jnp.max(jnp.where(mask, py, neg))
            nlz = jnp.max(jnp.where(mask, pz, neg))
            oi_ref[0, pl.ds(i, 1), 0:1] = nxt.reshape(1, 1)
            op_ref[0, pl.ds(i, 1), 0:1] = nlx.reshape(1, 1)
            op_ref[0, pl.ds(i, 1), 1:2] = nly.reshape(1, 1)
            op_ref[0, pl.ds(i, 1), 2:3] = nlz.reshape(1, 1)
            return dist, nlx, nly, nlz

        init = (jnp.full((8, n8), jnp.inf, F32),
                p_ref[0, 0, 0, 0], p_ref[0, 1, 0, 0], p_ref[0, 2, 0, 0])
        lax.fori_loop(1, m, step, init)

    return pl.pallas_call(
        body,
        grid=(2,),
        in_specs=[pl.BlockSpec((1, 3, 8, n8), lambda b: (b, 0, 0, 0))],
        out_specs=[
            pl.BlockSpec((1, m, 1), lambda b: (b, 0, 0)),
            pl.BlockSpec((1, m, 3), lambda b: (b, 0, 0)),
        ],
        out_shape=[
            jax.ShapeDtypeStruct((2, m, 1), I32),
            jax.ShapeDtypeStruct((2, m, 3), F32),
        ],
    )(posr)


def _down_max_call(gh, m, c):
    blk = min(m, 256)

    def body(g_ref, o_ref):
        g3 = g_ref[0]
        acc = g3[0]
        for k in range(1, K):
            acc = jnp.maximum(acc, g3[k])
        o_ref[0] = acc

    return pl.pallas_call(
        body,
        grid=(2, m // blk),
        in_specs=[pl.BlockSpec((1, K, blk, c), lambda b, i: (b, 0, i, 0))],
        out_specs=pl.BlockSpec((1, blk, c), lambda b, i: (b, i, 0)),
        out_shape=jax.ShapeDtypeStruct((2, m, c), F32),
    )(gh)


def _up_interp_call(gx, pj, pos, lbrx, n, c):
    blk = min(n, 512)

    def body(gx_ref, pj_ref, pp_ref, lx_ref, o_ref):
        pd = pp_ref[0]
        ws = None
        acc = None
        for k in range(3):
            pj = pj_ref[0, k]
            dd = pd - pj
            d = (dd[:, 0:1] * dd[:, 0:1] + dd[:, 1:2] * dd[:, 1:2]
                 + dd[:, 2:3] * dd[:, 2:3])
            w = 1.0 / jnp.maximum(d, 1e-16)
            term = gx_ref[0, k] * w
            ws = w if ws is None else ws + w
            acc = term if acc is None else acc + term
        o_ref[0] = lx_ref[0] + acc / ws

    return pl.pallas_call(
        body,
        grid=(2, n // blk),
        in_specs=[
            pl.BlockSpec((1, 3, blk, c), lambda b, i: (b, 0, i, 0)),
            pl.BlockSpec((1, 3, blk, 3), lambda b, i: (b, 0, i, 0)),
            pl.BlockSpec((1, blk, 3), lambda b, i: (b, i, 0)),
            pl.BlockSpec((1, blk, c), lambda b, i: (b, i, 0)),
        ],
        out_specs=pl.BlockSpec((1, blk, c), lambda b, i: (b, i, 0)),
        out_shape=jax.ShapeDtypeStruct((2, n, c), F32),
    )(gx, pj, pos, lbrx)


def _head_call(x, hp):
    r = x.shape[0]
    blk = min(r, 512)

    def ln(h, g, b):
        mu = jnp.mean(h, axis=-1, keepdims=True)
        var = jnp.mean((h - mu) ** 2, axis=-1, keepdims=True)
        return (h - mu) / jnp.sqrt(var + 1e-5) * g + b

    def body(x_ref, c1w, c1b, c2w, c2b, c3w, c3b, g1, b1, g2, b2, o_ref):
        h = _matmul(x_ref[...], c1w[...]) + c1b[...]
        h = ln(h, g1[...], b1[...])
        h = _matmul(h, c2w[...]) + c2b[...]
        h = ln(h, g2[...], b2[...])
        h = _matmul(h, c3w[...]) + c3b[...]
        sig = 1.0 / (1.0 + jnp.exp(-h))
        o_ref[...] = sig * 2.0 - 1.0

    return pl.pallas_call(
        body,
        grid=(r // blk,),
        in_specs=[
            pl.BlockSpec((blk, 128), lambda i: (i, 0)),
            pl.BlockSpec((128, 32), lambda i: (0, 0)),
            pl.BlockSpec((1, 32), lambda i: (0, 0)),
            pl.BlockSpec((32, 32), lambda i: (0, 0)),
            pl.BlockSpec((1, 32), lambda i: (0, 0)),
            pl.BlockSpec((32, 3), lambda i: (0, 0)),
            pl.BlockSpec((1, 3), lambda i: (0, 0)),
            pl.BlockSpec((1, 32), lambda i: (0, 0)),
            pl.BlockSpec((1, 32), lambda i: (0, 0)),
            pl.BlockSpec((1, 32), lambda i: (0, 0)),
            pl.BlockSpec((1, 32), lambda i: (0, 0)),
        ],
        out_specs=pl.BlockSpec((blk, 3), lambda i: (i, 0)),
        out_shape=jax.ShapeDtypeStruct((r, 3), F32),
    )(x, hp['c1W'], hp['c1b'].reshape(1, 32), hp['c2W'],
      hp['c2b'].reshape(1, 32), hp['c3W'], hp['c3b'].reshape(1, 3),
      hp['ln1_g'].reshape(1, 32), hp['ln1_b'].reshape(1, 32),
      hp['ln2_g'].reshape(1, 32), hp['ln2_b'].reshape(1, 32))


def _pos_r(pos, n):
    return jnp.transpose(pos, (0, 2, 1)).reshape(2, 3, 8, n // 8)


def _transition_down(x_flat, pos, n, m, p):
    idc, sub_pos = _fps_call(_pos_r(pos, n), m)
    base_t = jnp.transpose(pos, (0, 2, 1))
    nbr = _knn_call(sub_pos, base_t, K, False)
    h = _lbr_call(x_flat, p)
    idx = _kmajor_idx(nbr, n)
    gh = _sc_gather(h, idx).reshape(2, K, m, h.shape[1])
    out = _down_max_call(gh, m, h.shape[1])
    return out.reshape(2 * m, h.shape[1]), sub_pos


def _transition_up(x_flat, xsub_flat, pos, pos_sub, n, m, p_sub, p_mlp):
    xs = _lbr_call(xsub_flat, p_sub)
    c = xs.shape[1]
    sub_t = jnp.transpose(pos_sub, (0, 2, 1))
    nbr, pj = _knn_call(pos, sub_t, 3, False, want_pj=True)
    offs = (jnp.arange(2, dtype=I32) * m)[:, None, None]
    idx = jnp.transpose(nbr + offs, (0, 2, 1)).reshape(-1)
    gx = _sc_gather(xs, idx).reshape(2, 3, n, c)
    lbrx = _lbr_call(x_flat, p_mlp)
    out = _up_interp_call(gx, pj, pos, lbrx.reshape(2, n, c), n, c)
    return out.reshape(2 * n, c)


def kernel(cloud, params):
    p = params
    n0 = cloud.shape[1]
    n1, n2 = n0 // 4, n0 // 16
    pos0 = cloud
    pos0_flat = pos0.reshape(2 * n0, 3)
    pos0_pad = jnp.pad(pos0_flat, ((0, 0), (0, 125)))
    pos0_t = jnp.transpose(pos0, (0, 2, 1))

    x = _lbr_call(pos0_flat, p['mlp_in'])
    nbr0 = _knn_call(pos0, pos0_t, K, True)
    idx0 = _kmajor_idx(nbr0, n0)
    gp0 = _sc_gather(pos0_pad, idx0).reshape(2, K, n0, 128)
    x0 = _transformer_block(x, pos0, gp0, idx0, n0, 128, p['t_in'])

    x1, pos1 = _transition_down(x0, pos0, n0, n1, p['td0'])
    pos1_t = jnp.transpose(pos1, (0, 2, 1))
    pos1_pad = jnp.pad(pos1.reshape(2 * n1, 3), ((0, 0), (0, 125)))
    nbr1 = _knn_call(pos1, pos1_t, K, True)
    idx1 = _kmajor_idx(nbr1, n1)
    gp1 = _sc_gather(pos1_pad, idx1).reshape(2, K, n1, 128)
    x1 = _transformer_block(x1, pos1, gp1, idx1, n1, 256, p['t_d0'])

    x2, pos2 = _transition_down(x1, pos1, n1, n2, p['td1'])
    pos2_t = jnp.transpose(pos2, (0, 2, 1))
    pos2_pad = jnp.pad(pos2.reshape(2 * n2, 3), ((0, 0), (0, 125)))
    nbr2 = _knn_call(pos2, pos2_t, K, True)
    idx2 = _kmajor_idx(nbr2, n2)
    gp2 = _sc_gather(pos2_pad, idx2).reshape(2, K, n2, 128)
    x2 = _transformer_block(x2, pos2, gp2, idx2, n2, 512, p['t_d1'])

    x2 = _dense_relu_call(x2, p['summit']['W'], p['summit']['b'])
    x2 = _transformer_block(x2, pos2, gp2, idx2, n2, 512, p['t_sum'])

    xu1 = _transition_up(x1, x2, pos1, pos2, n1, n2,
                         p['tu1_sub'], p['tu1_mlp'])
    xu1 = _transformer_block(xu1, pos1, gp1, idx1, n1, 256, p['t_u1'])

    xu0 = _transition_up(x0, xu1, pos0, pos1, n0, n1,
                         p['tu0_sub'], p['tu0_mlp'])
    xu0 = _transformer_block(xu0, pos0, gp0, idx0, n0, 128, p['t_u0'])

    out = _head_call(xu0, p['head'])
    return out.reshape(2, n0, 3)


# speedup vs baseline: 1.3484x; 1.0831x over previous
import functools

import jax
import jax.numpy as jnp
from jax import lax
from jax.experimental import pallas as pl
from jax.experimental.pallas import tpu as pltpu
from jax.experimental.pallas import tpu_sc as plsc

F32 = jnp.float32
I32 = jnp.int32
K = 32
_NC, _NS = 2, 16
_NW = _NC * _NS
_INV_LBR = 1.0 / (1.0 + 1e-5) ** 0.5


def _pick_chunk(b_per_w, d):
    budget = 360 * 1024 // (4 * d)
    c = b_per_w
    while c > budget or c % 8 != 0:
        if c % 2 != 0:
            return 8
        c //= 2
    return max(c, 8)


@functools.lru_cache(maxsize=None)
def _sc_gather_fn(v_rows, row_shape, b_total, dtype):
    b_per_w = b_total // _NW
    esize = jnp.dtype(dtype).itemsize
    row_elems = 1
    for s in row_shape:
        row_elems *= s
    chunk = _pick_chunk(b_per_w, row_elems * esize // 4)
    iters = b_per_w // chunk
    mesh = plsc.VectorSubcoreMesh(core_axis_name="c", subcore_axis_name="s")

    @functools.partial(
        pl.kernel,
        out_type=jax.ShapeDtypeStruct((b_total,) + row_shape, dtype),
        mesh=mesh,
        scratch_types=[
            pltpu.VMEM((chunk,), I32),
            pltpu.VMEM((chunk,) + row_shape, dtype),
            pltpu.SemaphoreType.DMA,
        ],
    )
    def gather_kernel(table_hbm, idx_hbm, out_hbm, idx_v, rows_v, sem):
        wid = lax.axis_index("s") * _NC + lax.axis_index("c")
        base0 = wid * b_per_w
        for t in range(iters):
            base = base0 + t * chunk
            pltpu.sync_copy(idx_hbm.at[pl.ds(base, chunk)], idx_v)
            pltpu.async_copy(table_hbm.at[idx_v], rows_v, sem).wait()
            pltpu.sync_copy(rows_v, out_hbm.at[pl.ds(base, chunk)])

    return gather_kernel


def _sc_gather(table, idx):
    v_rows = table.shape[0]
    row_shape = table.shape[1:]
    (b_total,) = idx.shape
    assert b_total % (8 * _NW) == 0, (table.shape, idx.shape)
    return _sc_gather_fn(v_rows, row_shape, b_total,
                         jnp.dtype(table.dtype).name)(table, idx)


def _matmul(a, b):
    return jnp.dot(a, b, preferred_element_type=F32)


def _lbr_call(x, p):
    r, ci = x.shape
    co = p['W'].shape[1]
    blk = min(r, 512)

    def body(x_ref, w_ref, b_ref, g_ref, be_ref, o_ref):
        h = _matmul(x_ref[...], w_ref[...]) + b_ref[...]
        h = h * _INV_LBR * g_ref[...] + be_ref[...]
        o_ref[...] = jnp.maximum(h, 0.0)

    return pl.pallas_call(
        body,
        grid=(r // blk,),
        in_specs=[
            pl.BlockSpec((blk, ci), lambda i: (i, 0)),
            pl.BlockSpec((ci, co), lambda i: (0, 0)),
            pl.BlockSpec((1, co), lambda i: (0, 0)),
            pl.BlockSpec((1, co), lambda i: (0, 0)),
            pl.BlockSpec((1, co), lambda i: (0, 0)),
        ],
        out_specs=pl.BlockSpec((blk, co), lambda i: (i, 0)),
        out_shape=jax.ShapeDtypeStruct((r, co), F32),
    )(x, p['W'], p['b'].reshape(1, co), p['g'].reshape(1, co),
      p['be'].reshape(1, co))


def _dense_relu_call(x, w, b):
    r, ci = x.shape
    co = w.shape[1]
    blk = min(r, 512)

    def body(x_ref, w_ref, b_ref, o_ref):
        o_ref[...] = jnp.maximum(_matmul(x_ref[...], w_ref[...]) + b_ref[...],
                                 0.0)

    return pl.pallas_call(
        body,
        grid=(r // blk,),
        in_specs=[
            pl.BlockSpec((blk, ci), lambda i: (i, 0)),
            pl.BlockSpec((ci, co), lambda i: (0, 0)),
            pl.BlockSpec((1, co), lambda i: (0, 0)),
        ],
        out_specs=pl.BlockSpec((blk, co), lambda i: (i, 0)),
        out_shape=jax.ShapeDtypeStruct((r, co), F32),
    )(x, w, b.reshape(1, co))


def _rtne16(bits):
    return bits + 0x7FFF + jnp.bitwise_and(jnp.right_shift(bits, 16), 1)


def _node_transform_call(x, p):
    r, c = x.shape
    blk = min(r, 256)

    def body(x_ref, wi_ref, bi_ref, w_ref, ws_ref, wd_ref,
             va_ref, ad_ref):
        x1 = jnp.maximum(_matmul(x_ref[...], wi_ref[...]) + bi_ref[...], 0.0)
        vb = lax.bitcast_convert_type(_matmul(x1, w_ref[...]), I32)
        ab = lax.bitcast_convert_type(_matmul(x1, ws_ref[...]), I32)
        lo = jnp.bitwise_and(jnp.right_shift(_rtne16(vb), 16), 0xFFFF)
        hi = jnp.bitwise_and(_rtne16(ab), jnp.int32(-65536))
        va_ref[...] = jnp.bitwise_or(lo, hi)
        ad_ref[...] = _matmul(x1, wd_ref[...])

    outs = pl.pallas_call(
        body,
        grid=(r // blk,),
        in_specs=[
            pl.BlockSpec((blk, c), lambda i: (i, 0)),
            pl.BlockSpec((c, c), lambda i: (0, 0)),
            pl.BlockSpec((1, c), lambda i: (0, 0)),
            pl.BlockSpec((c, c), lambda i: (0, 0)),
            pl.BlockSpec((c, c), lambda i: (0, 0)),
            pl.BlockSpec((c, c), lambda i: (0, 0)),
        ],
        out_specs=[pl.BlockSpec((blk, c), lambda i: (i, 0)),
                   pl.BlockSpec((blk, c), lambda i: (i, 0))],
        out_shape=[jax.ShapeDtypeStruct((r, c), I32),
                   jax.ShapeDtypeStruct((r, c), F32)],
    )(x, p['lin_in_W'], p['lin_in_b'].reshape(1, c),
      p['W'], p['W_src'], p['W_dst'])
    return outs


def _edge_attn_call(gva, gp, a_dst, pos, p, n, c):
    blk = max(2048 // c * 8, 8)
    blk = min(blk, n)
    while n % blk:
        blk //= 2

    def body(gva_ref, gp_ref, ad_ref, pp_ref,
             pw1_ref, pb1_ref, pw2_ref, pb2_ref,
             aw1_ref, ab1_ref, aw2_ref, ab2_ref,
             lo_ref, lb_ref, o_ref):
        kb = K * blk
        raw = gva_ref[0]
        gx3 = lax.bitcast_convert_type(jnp.left_shift(raw, 16), F32)
        ga3 = lax.bitcast_convert_type(
            jnp.bitwise_and(raw, jnp.int32(-65536)), F32)
        gp3 = gp_ref[0][:, :, 0:3]
        ad = ad_ref[0]
        pd = pp_ref[0]

        rel3 = jnp.broadcast_to(pd[None], (K, blk, 3)) - gp3
        rel = rel3.reshape(kb, 3)
        h = (rel[:, 0:1] * pw1_ref[0:1, :] + rel[:, 1:2] * pw1_ref[1:2, :]
             + rel[:, 2:3] * pw1_ref[2:3, :]) + pb1_ref[...]
        h = jnp.maximum(h, 0.0)
        delta = jnp.maximum(_matmul(h, pw2_ref[...]) + pb2_ref[...], 0.0)
        delta3 = delta.reshape(K, blk, c)

        alpha0 = (jnp.broadcast_to(ad[None], (K, blk, c)) - ga3
                  + delta3).reshape(kb, c)
        t = jnp.maximum(_matmul(alpha0, aw1_ref[...]) + ab1_ref[...], 0.0)
        alpha = jnp.maximum(_matmul(t, aw2_ref[...]) + ab2_ref[...], 0.0)
        alpha3 = alpha.reshape(K, blk, c)

        m = alpha3[0]
        for k in range(1, K):
            m = jnp.maximum(m, alpha3[k])
        e3 = jnp.exp(alpha3 - m[None])
        s = e3[0]
        for k in range(1, K):
            s = s + e3[k]
        w3 = e3 / s[None]
        acc = w3[0] * (gx3[0] + delta3[0])
        for k in range(1, K):
            acc = acc + w3[k] * (gx3[k] + delta3[k])

        o_ref[0] = jnp.maximum(_matmul(acc, lo_ref[...]) + lb_ref[...], 0.0)

    return pl.pallas_call(
        body,
        grid=(2, n // blk),
        in_specs=[
            pl.BlockSpec((1, K, blk, c), lambda b, i: (b, 0, i, 0)),
            pl.BlockSpec((1, K, blk, 128), lambda b, i: (b, 0, i, 0)),
            pl.BlockSpec((1, blk, c), lambda b, i: (b, i, 0)),
            pl.BlockSpec((1, blk, 3), lambda b, i: (b, i, 0)),
            pl.BlockSpec((3, 64), lambda b, i: (0, 0)),
            pl.BlockSpec((1, 64), lambda b, i: (0, 0)),
            pl.BlockSpec((64, c), lambda b, i: (0, 0)),
            pl.BlockSpec((1, c), lambda b, i: (0, 0)),
            pl.BlockSpec((c, 64), lambda b, i: (0, 0)),
            pl.BlockSpec((1, 64), lambda b, i: (0, 0)),
            pl.BlockSpec((64, c), lambda b, i: (0, 0)),
            pl.BlockSpec((1, c), lambda b, i: (0, 0)),
            pl.BlockSpec((c, c), lambda b, i: (0, 0)),
            pl.BlockSpec((1, c), lambda b, i: (0, 0)),
        ],
        out_specs=pl.BlockSpec((1, blk, c), lambda b, i: (b, i, 0)),
        out_shape=jax.ShapeDtypeStruct((2, n, c), F32),
    )(gva, gp, a_dst, pos,
      p['pos_W1'], p['pos_b1'].reshape(1, 64), p['pos_W2'],
      p['pos_b2'].reshape(1, c),
      p['attn_W1'], p['attn_b1'].reshape(1, 64), p['attn_W2'],
      p['attn_b2'].reshape(1, c),
      p['lin_out_W'], p['lin_out_b'].reshape(1, c))


def _transformer_block(x_flat, pos, gp, idx, n, c, p):
    va, a_dst = _node_transform_call(x_flat, p)
    gva = _sc_gather(va, idx).reshape(2, K, n, c)
    out = _edge_attn_call(gva, gp, a_dst.reshape(2, n, c), pos, p, n, c)
    return out.reshape(2 * n, c)


def _kmajor_idx(nbr, n):
    offs = (jnp.arange(2, dtype=I32) * n)[:, None, None]
    return jnp.transpose(nbr + offs, (0, 2, 1)).reshape(-1)


def _knn_call(query, base_t, kk, exclude_self, want_pj=False):
    _, q, _ = query.shape
    nb = base_t.shape[2]
    bq = min(q, 256)
    inf = float('inf')
    big = 2 ** 30

    def body(q_ref, b_ref, o_ref, *pj_ref):
        qs = q_ref[0]
        bx = b_ref[0, 0:1, :]
        by = b_ref[0, 1:2, :]
        bz = b_ref[0, 2:3, :]
        dx = qs[:, 0:1] - bx
        dy = qs[:, 1:2] - by
        dz = qs[:, 2:3] - bz
        d = dx * dx + dy * dy + dz * dz
        col = lax.broadcasted_iota(I32, (bq, nb), 1)
        if exclude_self:
            row = (lax.broadcasted_iota(I32, (bq, nb), 0)
                   + pl.program_id(1) * bq)
            d = jnp.where(row == col, inf, d)
        cols = []
        for t in range(kk):
            mval = jnp.min(d, axis=1, keepdims=True)
            sel = jnp.where(d == mval, col, big)
            midx = jnp.min(sel, axis=1, keepdims=True)
            cols.append(midx)
            m2 = col == midx
            if want_pj:
                pj = pj_ref[0]
                pj[0, t, :, 0:1] = jnp.min(
                    jnp.where(m2, bx, inf), axis=1, keepdims=True)
                pj[0, t, :, 1:2] = jnp.min(
                    jnp.where(m2, by, inf), axis=1, keepdims=True)
                pj[0, t, :, 2:3] = jnp.min(
                    jnp.where(m2, bz, inf), axis=1, keepdims=True)
            d = jnp.where(m2, inf, d)
        o_ref[0] = jnp.concatenate(cols, axis=1)

    out_specs = [pl.BlockSpec((1, bq, kk), lambda b, i: (b, i, 0))]
    out_shape = [jax.ShapeDtypeStruct((2, q, kk), I32)]
    if want_pj:
        out_specs.append(pl.BlockSpec((1, kk, bq, 3), lambda b, i: (b, 0, i, 0)))
        out_shape.append(jax.ShapeDtypeStruct((2, kk, q, 3), F32))
    res = pl.pallas_call(
        body,
        grid=(2, q // bq),
        in_specs=[
            pl.BlockSpec((1, bq, 3), lambda b, i: (b, i, 0)),
            pl.BlockSpec((1, 3, nb), lambda b, i: (b, 0, 0)),
        ],
        out_specs=out_specs,
        out_shape=out_shape,
    )(query, base_t)
    return res if want_pj else res[0]


def _fps_call(posr, m):
    n8 = posr.shape[3]
    big = 2 ** 30
    neg = -1e30

    def body(p_ref, oi_ref, op_ref):
        px = p_ref[0, 0]
        py = p_ref[0, 1]
        pz = p_ref[0, 2]
        fiota = (lax.broadcasted_iota(I32, (8, n8), 0) * n8
                 + lax.broadcasted_iota(I32, (8, n8), 1))
        oi_ref[0, 0:1, 0:1] = jnp.zeros((1, 1), I32)
        op_ref[0, 0:1, 0:1] = p_ref[0, 0, 0, 0].reshape(1, 1)
        op_ref[0, 0:1, 1:2] = p_ref[0, 1, 0, 0].reshape(1, 1)
        op_ref[0, 0:1, 2:3] = p_ref[0, 2, 0, 0].reshape(1, 1)

        def step(i, carry):
            dist, lx, ly, lz = carry
            dxx = px - lx
            dyy = py - ly
            dzz = pz - lz
            d = dxx * dxx + dyy * dyy + dzz * dzz
            dist = jnp.minimum(dist, d)
            mx = jnp.max(dist)
            sel = jnp.where(dist == mx, fiota, big)
            nxt = jnp.min(sel)
            mask = fiota == nxt
            nlx = jnp.max(jnp.where(mask, px, neg))
            nly = jnp.max(jnp.where(mask, py, neg))
            nlz = jnp.max(jnp.where(mask, pz, neg))
            oi_ref[0, pl.ds(i, 1), 0:1] = nxt.reshape(1, 1)
            op_ref[0, pl.ds(i, 1), 0:1] = nlx.reshape(1, 1)
            op_ref[0, pl.ds(i, 1), 1:2] = nly.reshape(1, 1)
            op_ref[0, pl.ds(i, 1), 2:3] = nlz.reshape(1, 1)
            return dist, nlx, nly, nlz

        init = (jnp.full((8, n8), jnp.inf, F32),
                p_ref[0, 0, 0, 0], p_ref[0, 1, 0, 0], p_ref[0, 2, 0, 0])
        lax.fori_loop(1, m, step, init)

    return pl.pallas_call(
        body,
        grid=(2,),
        in_specs=[pl.BlockSpec((1, 3, 8, n8), lambda b: (b, 0, 0, 0))],
        out_specs=[
            pl.BlockSpec((1, m, 1), lambda b: (b, 0, 0)),
            pl.BlockSpec((1, m, 3), lambda b: (b, 0, 0)),
        ],
        out_shape=[
            jax.ShapeDtypeStruct((2, m, 1), I32),
            jax.ShapeDtypeStruct((2, m, 3), F32),
        ],
    )(posr)


def _down_max_call(gh, m, c):
    blk = min(m, 256)

    def body(g_ref, o_ref):
        g3 = g_ref[0]
        acc = g3[0]
        for k in range(1, K):
            acc = jnp.maximum(acc, g3[k])
        o_ref[0] = acc

    return pl.pallas_call(
        body,
        grid=(2, m // blk),
        in_specs=[pl.BlockSpec((1, K, blk, c), lambda b, i: (b, 0, i, 0))],
        out_specs=pl.BlockSpec((1, blk, c), lambda b, i: (b, i, 0)),
        out_shape=jax.ShapeDtypeStruct((2, m, c), F32),
    )(gh)


def _up_interp_call(gx, pj, pos, lbrx, n, c):
    blk = min(n, 512)

    def body(gx_ref, pj_ref, pp_ref, lx_ref, o_ref):
        pd = pp_ref[0]
        ws = None
        acc = None
        for k in range(3):
            pj = pj_ref[0, k]
            dd = pd - pj
            d = (dd[:, 0:1] * dd[:, 0:1] + dd[:, 1:2] * dd[:, 1:2]
                 + dd[:, 2:3] * dd[:, 2:3])
            w = 1.0 / jnp.maximum(d, 1e-16)
            term = gx_ref[0, k] * w
            ws = w if ws is None else ws + w
            acc = term if acc is None else acc + term
        o_ref[0] = lx_ref[0] + acc / ws

    return pl.pallas_call(
        body,
        grid=(2, n // blk),
        in_specs=[
            pl.BlockSpec((1, 3, blk, c), lambda b, i: (b, 0, i, 0)),
            pl.BlockSpec((1, 3, blk, 3), lambda b, i: (b, 0, i, 0)),
            pl.BlockSpec((1, blk, 3), lambda b, i: (b, i, 0)),
            pl.BlockSpec((1, blk, c), lambda b, i: (b, i, 0)),
        ],
        out_specs=pl.BlockSpec((1, blk, c), lambda b, i: (b, i, 0)),
        out_shape=jax.ShapeDtypeStruct((2, n, c), F32),
    )(gx, pj, pos, lbrx)


def _head_call(x, hp):
    r = x.shape[0]
    blk = min(r, 512)

    def ln(h, g, b):
        mu = jnp.mean(h, axis=-1, keepdims=True)
        var = jnp.mean((h - mu) ** 2, axis=-1, keepdims=True)
        return (h - mu) / jnp.sqrt(var + 1e-5) * g + b

    def body(x_ref, c1w, c1b, c2w, c2b, c3w, c3b, g1, b1, g2, b2, o_ref):
        h = _matmul(x_ref[...], c1w[...]) + c1b[...]
        h = ln(h, g1[...], b1[...])
        h = _matmul(h, c2w[...]) + c2b[...]
        h = ln(h, g2[...], b2[...])
        h = _matmul(h, c3w[...]) + c3b[...]
        sig = 1.0 / (1.0 + jnp.exp(-h))
        o_ref[...] = sig * 2.0 - 1.0

    return pl.pallas_call(
        body,
        grid=(r // blk,),
        in_specs=[
            pl.BlockSpec((blk, 128), lambda i: (i, 0)),
            pl.BlockSpec((128, 32), lambda i: (0, 0)),
            pl.BlockSpec((1, 32), lambda i: (0, 0)),
            pl.BlockSpec((32, 32), lambda i: (0, 0)),
            pl.BlockSpec((1, 32), lambda i: (0, 0)),
            pl.BlockSpec((32, 3), lambda i: (0, 0)),
            pl.BlockSpec((1, 3), lambda i: (0, 0)),
            pl.BlockSpec((1, 32), lambda i: (0, 0)),
            pl.BlockSpec((1, 32), lambda i: (0, 0)),
            pl.BlockSpec((1, 32), lambda i: (0, 0)),
            pl.BlockSpec((1, 32), lambda i: (0, 0)),
        ],
        out_specs=pl.BlockSpec((blk, 3), lambda i: (i, 0)),
        out_shape=jax.ShapeDtypeStruct((r, 3), F32),
    )(x, hp['c1W'], hp['c1b'].reshape(1, 32), hp['c2W'],
      hp['c2b'].reshape(1, 32), hp['c3W'], hp['c3b'].reshape(1, 3),
      hp['ln1_g'].reshape(1, 32), hp['ln1_b'].reshape(1, 32),
      hp['ln2_g'].reshape(1, 32), hp['ln2_b'].reshape(1, 32))


def _pos_r(pos, n):
    return jnp.transpose(pos, (0, 2, 1)).reshape(2, 3, 8, n // 8)


def _transition_down(x_flat, pos, n, m, p):
    idc, sub_pos = _fps_call(_pos_r(pos, n), m)
    base_t = jnp.transpose(pos, (0, 2, 1))
    nbr = _knn_call(sub_pos, base_t, K, False)
    h = _lbr_call(x_flat, p)
    idx = _kmajor_idx(nbr, n)
    gh = _sc_gather(h, idx).reshape(2, K, m, h.shape[1])
    out = _down_max_call(gh, m, h.shape[1])
    return out.reshape(2 * m, h.shape[1]), sub_pos


def _transition_up(x_flat, xsub_flat, pos, pos_sub, n, m, p_sub, p_mlp):
    xs = _lbr_call(xsub_flat, p_sub)
    c = xs.shape[1]
    sub_t = jnp.transpose(pos_sub, (0, 2, 1))
    nbr, pj = _knn_call(pos, sub_t, 3, False, want_pj=True)
    offs = (jnp.arange(2, dtype=I32) * m)[:, None, None]
    idx = jnp.transpose(nbr + offs, (0, 2, 1)).reshape(-1)
    gx = _sc_gather(xs, idx).reshape(2, 3, n, c)
    lbrx = _lbr_call(x_flat, p_mlp)
    out = _up_interp_call(gx, pj, pos, lbrx.reshape(2, n, c), n, c)
    return out.reshape(2 * n, c)


def kernel(cloud, params):
    p = params
    n0 = cloud.shape[1]
    n1, n2 = n0 // 4, n0 // 16
    pos0 = cloud
    pos0_flat = pos0.reshape(2 * n0, 3)
    pos0_pad = jnp.pad(pos0_flat, ((0, 0), (0, 125)))
    pos0_t = jnp.transpose(pos0, (0, 2, 1))

    x = _lbr_call(pos0_flat, p['mlp_in'])
    nbr0 = _knn_call(pos0, pos0_t, K, True)
    idx0 = _kmajor_idx(nbr0, n0)
    gp0 = _sc_gather(pos0_pad, idx0).reshape(2, K, n0, 128)
    x0 = _transformer_block(x, pos0, gp0, idx0, n0, 128, p['t_in'])

    x1, pos1 = _transition_down(x0, pos0, n0, n1, p['td0'])
    pos1_t = jnp.transpose(pos1, (0, 2, 1))
    pos1_pad = jnp.pad(pos1.reshape(2 * n1, 3), ((0, 0), (0, 125)))
    nbr1 = _knn_call(pos1, pos1_t, K, True)
    idx1 = _kmajor_idx(nbr1, n1)
    gp1 = _sc_gather(pos1_pad, idx1).reshape(2, K, n1, 128)
    x1 = _transformer_block(x1, pos1, gp1, idx1, n1, 256, p['t_d0'])

    x2, pos2 = _transition_down(x1, pos1, n1, n2, p['td1'])
    pos2_t = jnp.transpose(pos2, (0, 2, 1))
    pos2_pad = jnp.pad(pos2.reshape(2 * n2, 3), ((0, 0), (0, 125)))
    nbr2 = _knn_call(pos2, pos2_t, K, True)
    idx2 = _kmajor_idx(nbr2, n2)
    gp2 = _sc_gather(pos2_pad, idx2).reshape(2, K, n2, 128)
    x2 = _transformer_block(x2, pos2, gp2, idx2, n2, 512, p['t_d1'])

    x2 = _dense_relu_call(x2, p['summit']['W'], p['summit']['b'])
    x2 = _transformer_block(x2, pos2, gp2, idx2, n2, 512, p['t_sum'])

    xu1 = _transition_up(x1, x2, pos1, pos2, n1, n2,
                         p['tu1_sub'], p['tu1_mlp'])
    xu1 = _transformer_block(xu1, pos1, gp1, idx1, n1, 256, p['t_u1'])

    xu0 = _transition_up(x0, xu1, pos0, pos1, n0, n1,
                         p['tu0_sub'], p['tu0_mlp'])
    xu0 = _transformer_block(xu0, pos0, gp0, idx0, n0, 128, p['t_u0'])

    out = _head_call(xu0, p['head'])
    return out.reshape(2, n0, 3)


# speedup vs baseline: 1.3515x; 1.0023x over previous
import functools

import jax
import jax.numpy as jnp
from jax import lax
from jax.experimental import pallas as pl
from jax.experimental.pallas import tpu as pltpu
from jax.experimental.pallas import tpu_sc as plsc

F32 = jnp.float32
I32 = jnp.int32
K = 32
_NC, _NS = 2, 16
_NW = _NC * _NS
_INV_LBR = 1.0 / (1.0 + 1e-5) ** 0.5


def _pick_chunk(b_per_w, d):
    budget = 360 * 1024 // (4 * d)
    c = b_per_w
    while c > budget or c % 8 != 0:
        if c % 2 != 0:
            return 8
        c //= 2
    return max(c, 8)


@functools.lru_cache(maxsize=None)
def _sc_gather_fn(v_rows, row_shape, b_total, dtype):
    b_per_w = b_total // _NW
    esize = jnp.dtype(dtype).itemsize
    row_elems = 1
    for s in row_shape:
        row_elems *= s
    chunk = _pick_chunk(b_per_w, row_elems * esize // 4)
    iters = b_per_w // chunk
    mesh = plsc.VectorSubcoreMesh(core_axis_name="c", subcore_axis_name="s")

    @functools.partial(
        pl.kernel,
        out_type=jax.ShapeDtypeStruct((b_total,) + row_shape, dtype),
        mesh=mesh,
        scratch_types=[
            pltpu.VMEM((chunk,), I32),
            pltpu.VMEM((chunk,) + row_shape, dtype),
            pltpu.SemaphoreType.DMA,
        ],
    )
    def gather_kernel(table_hbm, idx_hbm, out_hbm, idx_v, rows_v, sem):
        wid = lax.axis_index("s") * _NC + lax.axis_index("c")
        base0 = wid * b_per_w
        for t in range(iters):
            base = base0 + t * chunk
            pltpu.sync_copy(idx_hbm.at[pl.ds(base, chunk)], idx_v)
            pltpu.async_copy(table_hbm.at[idx_v], rows_v, sem).wait()
            pltpu.sync_copy(rows_v, out_hbm.at[pl.ds(base, chunk)])

    return gather_kernel


def _sc_gather(table, idx):
    v_rows = table.shape[0]
    row_shape = table.shape[1:]
    (b_total,) = idx.shape
    assert b_total % (8 * _NW) == 0, (table.shape, idx.shape)
    return _sc_gather_fn(v_rows, row_shape, b_total,
                         jnp.dtype(table.dtype).name)(table, idx)


def _matmul(a, b):
    return jnp.dot(a, b, preferred_element_type=F32)


def _matmul_bf(a, b):
    return jnp.dot(a.astype(jnp.bfloat16), b.astype(jnp.bfloat16),
                   preferred_element_type=F32)


def _lbr_call(x, p):
    r, ci = x.shape
    co = p['W'].shape[1]
    blk = min(r, 512)

    def body(x_ref, w_ref, b_ref, g_ref, be_ref, o_ref):
        h = _matmul(x_ref[...], w_ref[...]) + b_ref[...]
        h = h * _INV_LBR * g_ref[...] + be_ref[...]
        o_ref[...] = jnp.maximum(h, 0.0)

    return pl.pallas_call(
        body,
        grid=(r // blk,),
        in_specs=[
            pl.BlockSpec((blk, ci), lambda i: (i, 0)),
            pl.BlockSpec((ci, co), lambda i: (0, 0)),
            pl.BlockSpec((1, co), lambda i: (0, 0)),
            pl.BlockSpec((1, co), lambda i: (0, 0)),
            pl.BlockSpec((1, co), lambda i: (0, 0)),
        ],
        out_specs=pl.BlockSpec((blk, co), lambda i: (i, 0)),
        out_shape=jax.ShapeDtypeStruct((r, co), F32),
    )(x, p['W'], p['b'].reshape(1, co), p['g'].reshape(1, co),
      p['be'].reshape(1, co))


def _dense_relu_call(x, w, b):
    r, ci = x.shape
    co = w.shape[1]
    blk = min(r, 512)

    def body(x_ref, w_ref, b_ref, o_ref):
        o_ref[...] = jnp.maximum(_matmul(x_ref[...], w_ref[...]) + b_ref[...],
                                 0.0)

    return pl.pallas_call(
        body,
        grid=(r // blk,),
        in_specs=[
            pl.BlockSpec((blk, ci), lambda i: (i, 0)),
            pl.BlockSpec((ci, co), lambda i: (0, 0)),
            pl.BlockSpec((1, co), lambda i: (0, 0)),
        ],
        out_specs=pl.BlockSpec((blk, co), lambda i: (i, 0)),
        out_shape=jax.ShapeDtypeStruct((r, co), F32),
    )(x, w, b.reshape(1, co))


def _rtne16(bits):
    return bits + 0x7FFF + jnp.bitwise_and(jnp.right_shift(bits, 16), 1)


def _node_transform_call(x, p):
    r, c = x.shape
    blk = min(r, 256)

    def body(x_ref, wi_ref, bi_ref, w_ref, ws_ref, wd_ref,
             va_ref, ad_ref):
        x1 = jnp.maximum(_matmul_bf(x_ref[...], wi_ref[...]) + bi_ref[...],
                         0.0)
        vb = lax.bitcast_convert_type(_matmul_bf(x1, w_ref[...]), I32)
        ab = lax.bitcast_convert_type(_matmul_bf(x1, ws_ref[...]), I32)
        lo = jnp.bitwise_and(jnp.right_shift(_rtne16(vb), 16), 0xFFFF)
        hi = jnp.bitwise_and(_rtne16(ab), jnp.int32(-65536))
        va_ref[...] = jnp.bitwise_or(lo, hi)
        ad_ref[...] = _matmul_bf(x1, wd_ref[...])

    outs = pl.pallas_call(
        body,
        grid=(r // blk,),
        in_specs=[
            pl.BlockSpec((blk, c), lambda i: (i, 0)),
            pl.BlockSpec((c, c), lambda i: (0, 0)),
            pl.BlockSpec((1, c), lambda i: (0, 0)),
            pl.BlockSpec((c, c), lambda i: (0, 0)),
            pl.BlockSpec((c, c), lambda i: (0, 0)),
            pl.BlockSpec((c, c), lambda i: (0, 0)),
        ],
        out_specs=[pl.BlockSpec((blk, c), lambda i: (i, 0)),
                   pl.BlockSpec((blk, c), lambda i: (i, 0))],
        out_shape=[jax.ShapeDtypeStruct((r, c), I32),
                   jax.ShapeDtypeStruct((r, c), F32)],
    )(x, p['lin_in_W'], p['lin_in_b'].reshape(1, c),
      p['W'], p['W_src'], p['W_dst'])
    return outs


def _edge_attn_call(gva, gp, a_dst, pos, p, n, c):
    blk = max(2048 // c * 8, 8)
    blk = min(blk, n)
    while n % blk:
        blk //= 2

    def body(gva_ref, gp_ref, ad_ref, pp_ref,
             pw1_ref, pb1_ref, pw2_ref, pb2_ref,
             aw1_ref, ab1_ref, aw2_ref, ab2_ref,
             lo_ref, lb_ref, o_ref):
        kb = K * blk
        raw = gva_ref[0]
        gx3 = lax.bitcast_convert_type(jnp.left_shift(raw, 16), F32)
        ga3 = lax.bitcast_convert_type(
            jnp.bitwise_and(raw, jnp.int32(-65536)), F32)
        gp3 = gp_ref[0][:, :, 0:3]
        ad = ad_ref[0]
        pd = pp_ref[0]

        rel3 = jnp.broadcast_to(pd[None], (K, blk, 3)) - gp3
        rel = rel3.reshape(kb, 3)
        h = (rel[:, 0:1] * pw1_ref[0:1, :] + rel[:, 1:2] * pw1_ref[1:2, :]
             + rel[:, 2:3] * pw1_ref[2:3, :]) + pb1_ref[...]
        h = jnp.maximum(h, 0.0)
        delta = jnp.maximum(_matmul_bf(h, pw2_ref[...]) + pb2_ref[...], 0.0)
        delta3 = delta.reshape(K, blk, c)

        alpha0 = (jnp.broadcast_to(ad[None], (K, blk, c)) - ga3
                  + delta3).reshape(kb, c)
        t = jnp.maximum(_matmul_bf(alpha0, aw1_ref[...]) + ab1_ref[...], 0.0)
        alpha = jnp.maximum(_matmul_bf(t, aw2_ref[...]) + ab2_ref[...], 0.0)
        alpha3 = alpha.reshape(K, blk, c)

        m = alpha3[0]
        for k in range(1, K):
            m = jnp.maximum(m, alpha3[k])
        e3 = jnp.exp(alpha3 - m[None])
        s = e3[0]
        for k in range(1, K):
            s = s + e3[k]
        w3 = e3 / s[None]
        acc = w3[0] * (gx3[0] + delta3[0])
        for k in range(1, K):
            acc = acc + w3[k] * (gx3[k] + delta3[k])

        o_ref[0] = jnp.maximum(_matmul_bf(acc, lo_ref[...]) + lb_ref[...], 0.0)

    return pl.pallas_call(
        body,
        grid=(2, n // blk),
        in_specs=[
            pl.BlockSpec((1, K, blk, c), lambda b, i: (b, 0, i, 0)),
            pl.BlockSpec((1, K, blk, 128), lambda b, i: (b, 0, i, 0)),
            pl.BlockSpec((1, blk, c), lambda b, i: (b, i, 0)),
            pl.BlockSpec((1, blk, 3), lambda b, i: (b, i, 0)),
            pl.BlockSpec((3, 64), lambda b, i: (0, 0)),
            pl.BlockSpec((1, 64), lambda b, i: (0, 0)),
            pl.BlockSpec((64, c), lambda b, i: (0, 0)),
            pl.BlockSpec((1, c), lambda b, i: (0, 0)),
            pl.BlockSpec((c, 64), lambda b, i: (0, 0)),
            pl.BlockSpec((1, 64), lambda b, i: (0, 0)),
            pl.BlockSpec((64, c), lambda b, i: (0, 0)),
            pl.BlockSpec((1, c), lambda b, i: (0, 0)),
            pl.BlockSpec((c, c), lambda b, i: (0, 0)),
            pl.BlockSpec((1, c), lambda b, i: (0, 0)),
        ],
        out_specs=pl.BlockSpec((1, blk, c), lambda b, i: (b, i, 0)),
        out_shape=jax.ShapeDtypeStruct((2, n, c), F32),
    )(gva, gp, a_dst, pos,
      p['pos_W1'], p['pos_b1'].reshape(1, 64), p['pos_W2'],
      p['pos_b2'].reshape(1, c),
      p['attn_W1'], p['attn_b1'].reshape(1, 64), p['attn_W2'],
      p['attn_b2'].reshape(1, c),
      p['lin_out_W'], p['lin_out_b'].reshape(1, c))


def _transformer_block(x_flat, pos, gp, idx, n, c, p):
    va, a_dst = _node_transform_call(x_flat, p)
    gva = _sc_gather(va, idx).reshape(2, K, n, c)
    out = _edge_attn_call(gva, gp, a_dst.reshape(2, n, c), pos, p, n, c)
    return out.reshape(2 * n, c)


def _kmajor_idx(nbr, n):
    offs = (jnp.arange(2, dtype=I32) * n)[:, None, None]
    return jnp.transpose(nbr + offs, (0, 2, 1)).reshape(-1)


def _knn_call(query, base_t, kk, exclude_self, want_pj=False):
    _, q, _ = query.shape
    nb = base_t.shape[2]
    bq = min(q, 256)
    inf = float('inf')
    big = 2 ** 30

    def body(q_ref, b_ref, o_ref, *pj_ref):
        qs = q_ref[0]
        bx = b_ref[0, 0:1, :]
        by = b_ref[0, 1:2, :]
        bz = b_ref[0, 2:3, :]
        dx = qs[:, 0:1] - bx
        dy = qs[:, 1:2] - by
        dz = qs[:, 2:3] - bz
        d = dx * dx + dy * dy + dz * dz
        col = lax.broadcasted_iota(I32, (bq, nb), 1)
        if exclude_self:
            row = (lax.broadcasted_iota(I32, (bq, nb), 0)
                   + pl.program_id(1) * bq)
            d = jnp.where(row == col, inf, d)
        cols = []
        for t in range(kk):
            mval = jnp.min(d, axis=1, keepdims=True)
            sel = jnp.where(d == mval, col, big)
            midx = jnp.min(sel, axis=1, keepdims=True)
            cols.append(midx)
            m2 = col == midx
            if want_pj:
                pj = pj_ref[0]
                pj[0, t, :, 0:1] = jnp.min(
                    jnp.where(m2, bx, inf), axis=1, keepdims=True)
                pj[0, t, :, 1:2] = jnp.min(
                    jnp.where(m2, by, inf), axis=1, keepdims=True)
                pj[0, t, :, 2:3] = jnp.min(
                    jnp.where(m2, bz, inf), axis=1, keepdims=True)
            d = jnp.where(m2, inf, d)
        o_ref[0] = jnp.concatenate(cols, axis=1)

    out_specs = [pl.BlockSpec((1, bq, kk), lambda b, i: (b, i, 0))]
    out_shape = [jax.ShapeDtypeStruct((2, q, kk), I32)]
    if want_pj:
        out_specs.append(pl.BlockSpec((1, kk, bq, 3), lambda b, i: (b, 0, i, 0)))
        out_shape.append(jax.ShapeDtypeStruct((2, kk, q, 3), F32))
    res = pl.pallas_call(
        body,
        grid=(2, q // bq),
        in_specs=[
            pl.BlockSpec((1, bq, 3), lambda b, i: (b, i, 0)),
            pl.BlockSpec((1, 3, nb), lambda b, i: (b, 0, 0)),
        ],
        out_specs=out_specs,
        out_shape=out_shape,
    )(query, base_t)
    return res if want_pj else res[0]


def _fps_call(posr, m):
    n8 = posr.shape[3]
    big = 2 ** 30
    neg = -1e30

    def body(p_ref, oi_ref, op_ref):
        px = p_ref[0, 0]
        py = p_ref[0, 1]
        pz = p_ref[0, 2]
        fiota = (lax.broadcasted_iota(I32, (8, n8), 0) * n8
                 + lax.broadcasted_iota(I32, (8, n8), 1))
        oi_ref[0, 0:1, 0:1] = jnp.zeros((1, 1), I32)
        op_ref[0, 0:1, 0:1] = p_ref[0, 0, 0, 0].reshape(1, 1)
        op_ref[0, 0:1, 1:2] = p_ref[0, 1, 0, 0].reshape(1, 1)
        op_ref[0, 0:1, 2:3] = p_ref[0, 2, 0, 0].reshape(1, 1)

        def step(i, carry):
            dist, lx, ly, lz = carry
            dxx = px - lx
            dyy = py - ly
            dzz = pz - lz
            d = dxx * dxx + dyy * dyy + dzz * dzz
            dist = jnp.minimum(dist, d)
            mx = jnp.max(dist)
            sel = jnp.where(dist == mx, fiota, big)
            nxt = jnp.min(sel)
            mask = fiota == nxt
            nlx = jnp.max(jnp.where(mask, px, neg))
            nly = jnp.max(jnp.where(mask, py, neg))
            nlz = jnp.max(jnp.where(mask, pz, neg))
            oi_ref[0, pl.ds(i, 1), 0:1] = nxt.reshape(1, 1)
            op_ref[0, pl.ds(i, 1), 0:1] = nlx.reshape(1, 1)
            op_ref[0, pl.ds(i, 1), 1:2] = nly.reshape(1, 1)
            op_ref[0, pl.ds(i, 1), 2:3] = nlz.reshape(1, 1)
            return dist, nlx, nly, nlz

        init = (jnp.full((8, n8), jnp.inf, F32),
                p_ref[0, 0, 0, 0], p_ref[0, 1, 0, 0], p_ref[0, 2, 0, 0])
        lax.fori_loop(1, m, step, init)

    return pl.pallas_call(
        body,
        grid=(2,),
        in_specs=[pl.BlockSpec((1, 3, 8, n8), lambda b: (b, 0, 0, 0))],
        out_specs=[
            pl.BlockSpec((1, m, 1), lambda b: (b, 0, 0)),
            pl.BlockSpec((1, m, 3), lambda b: (b, 0, 0)),
        ],
        out_shape=[
            jax.ShapeDtypeStruct((2, m, 1), I32),
            jax.ShapeDtypeStruct((2, m, 3), F32),
        ],
    )(posr)


def _down_max_call(gh, m, c):
    blk = min(m, 256)

    def body(g_ref, o_ref):
        g3 = g_ref[0]
        acc = g3[0]
        for k in range(1, K):
            acc = jnp.maximum(acc, g3[k])
        o_ref[0] = acc

    return pl.pallas_call(
        body,
        grid=(2, m // blk),
        in_specs=[pl.BlockSpec((1, K, blk, c), lambda b, i: (b, 0, i, 0))],
        out_specs=pl.BlockSpec((1, blk, c), lambda b, i: (b, i, 0)),
        out_shape=jax.ShapeDtypeStruct((2, m, c), F32),
    )(gh)


def _up_interp_call(gx, pj, pos, lbrx, n, c):
    blk = min(n, 512)

    def body(gx_ref, pj_ref, pp_ref, lx_ref, o_ref):
        pd = pp_ref[0]
        ws = None
        acc = None
        for k in range(3):
            pj = pj_ref[0, k]
            dd = pd - pj
            d = (dd[:, 0:1] * dd[:, 0:1] + dd[:, 1:2] * dd[:, 1:2]
                 + dd[:, 2:3] * dd[:, 2:3])
            w = 1.0 / jnp.maximum(d, 1e-16)
            term = gx_ref[0, k] * w
            ws = w if ws is None else ws + w
            acc = term if acc is None else acc + term
        o_ref[0] = lx_ref[0] + acc / ws

    return pl.pallas_call(
        body,
        grid=(2, n // blk),
        in_specs=[
            pl.BlockSpec((1, 3, blk, c), lambda b, i: (b, 0, i, 0)),
            pl.BlockSpec((1, 3, blk, 3), lambda b, i: (b, 0, i, 0)),
            pl.BlockSpec((1, blk, 3), lambda b, i: (b, i, 0)),
            pl.BlockSpec((1, blk, c), lambda b, i: (b, i, 0)),
        ],
        out_specs=pl.BlockSpec((1, blk, c), lambda b, i: (b, i, 0)),
        out_shape=jax.ShapeDtypeStruct((2, n, c), F32),
    )(gx, pj, pos, lbrx)


def _head_call(x, hp):
    r = x.shape[0]
    blk = min(r, 512)

    def ln(h, g, b):
        mu = jnp.mean(h, axis=-1, keepdims=True)
        var = jnp.mean((h - mu) ** 2, axis=-1, keepdims=True)
        return (h - mu) / jnp.sqrt(var + 1e-5) * g + b

    def body(x_ref, c1w, c1b, c2w, c2b, c3w, c3b, g1, b1, g2, b2, o_ref):
        h = _matmul(x_ref[...], c1w[...]) + c1b[...]
        h = ln(h, g1[...], b1[...])
        h = _matmul(h, c2w[...]) + c2b[...]
        h = ln(h, g2[...], b2[...])
        h = _matmul(h, c3w[...]) + c3b[...]
        sig = 1.0 / (1.0 + jnp.exp(-h))
        o_ref[...] = sig * 2.0 - 1.0

    return pl.pallas_call(
        body,
        grid=(r // blk,),
        in_specs=[
            pl.BlockSpec((blk, 128), lambda i: (i, 0)),
            pl.BlockSpec((128, 32), lambda i: (0, 0)),
            pl.BlockSpec((1, 32), lambda i: (0, 0)),
            pl.BlockSpec((32, 32), lambda i: (0, 0)),
            pl.BlockSpec((1, 32), lambda i: (0, 0)),
            pl.BlockSpec((32, 3), lambda i: (0, 0)),
            pl.BlockSpec((1, 3), lambda i: (0, 0)),
            pl.BlockSpec((1, 32), lambda i: (0, 0)),
            pl.BlockSpec((1, 32), lambda i: (0, 0)),
            pl.BlockSpec((1, 32), lambda i: (0, 0)),
            pl.BlockSpec((1, 32), lambda i: (0, 0)),
        ],
        out_specs=pl.BlockSpec((blk, 3), lambda i: (i, 0)),
        out_shape=jax.ShapeDtypeStruct((r, 3), F32),
    )(x, hp['c1W'], hp['c1b'].reshape(1, 32), hp['c2W'],
      hp['c2b'].reshape(1, 32), hp['c3W'], hp['c3b'].reshape(1, 3),
      hp['ln1_g'].reshape(1, 32), hp['ln1_b'].reshape(1, 32),
      hp['ln2_g'].reshape(1, 32), hp['ln2_b'].reshape(1, 32))


def _pos_r(pos, n):
    return jnp.transpose(pos, (0, 2, 1)).reshape(2, 3, 8, n // 8)


def _transition_down(x_flat, pos, n, m, p):
    idc, sub_pos = _fps_call(_pos_r(pos, n), m)
    base_t = jnp.transpose(pos, (0, 2, 1))
    nbr = _knn_call(sub_pos, base_t, K, False)
    h = _lbr_call(x_flat, p)
    idx = _kmajor_idx(nbr, n)
    gh = _sc_gather(h, idx).reshape(2, K, m, h.shape[1])
    out = _down_max_call(gh, m, h.shape[1])
    return out.reshape(2 * m, h.shape[1]), sub_pos


def _transition_up(x_flat, xsub_flat, pos, pos_sub, n, m, p_sub, p_mlp):
    xs = _lbr_call(xsub_flat, p_sub)
    c = xs.shape[1]
    sub_t = jnp.transpose(pos_sub, (0, 2, 1))
    nbr, pj = _knn_call(pos, sub_t, 3, False, want_pj=True)
    offs = (jnp.arange(2, dtype=I32) * m)[:, None, None]
    idx = jnp.transpose(nbr + offs, (0, 2, 1)).reshape(-1)
    gx = _sc_gather(xs, idx).reshape(2, 3, n, c)
    lbrx = _lbr_call(x_flat, p_mlp)
    out = _up_interp_call(gx, pj, pos, lbrx.reshape(2, n, c), n, c)
    return out.reshape(2 * n, c)


def kernel(cloud, params):
    p = params
    n0 = cloud.shape[1]
    n1, n2 = n0 // 4, n0 // 16
    pos0 = cloud
    pos0_flat = pos0.reshape(2 * n0, 3)
    pos0_pad = jnp.pad(pos0_flat, ((0, 0), (0, 125)))
    pos0_t = jnp.transpose(pos0, (0, 2, 1))

    x = _lbr_call(pos0_flat, p['mlp_in'])
    nbr0 = _knn_call(pos0, pos0_t, K, True)
    idx0 = _kmajor_idx(nbr0, n0)
    gp0 = _sc_gather(pos0_pad, idx0).reshape(2, K, n0, 128)
    x0 = _transformer_block(x, pos0, gp0, idx0, n0, 128, p['t_in'])

    x1, pos1 = _transition_down(x0, pos0, n0, n1, p['td0'])
    pos1_t = jnp.transpose(pos1, (0, 2, 1))
    pos1_pad = jnp.pad(pos1.reshape(2 * n1, 3), ((0, 0), (0, 125)))
    nbr1 = _knn_call(pos1, pos1_t, K, True)
    idx1 = _kmajor_idx(nbr1, n1)
    gp1 = _sc_gather(pos1_pad, idx1).reshape(2, K, n1, 128)
    x1 = _transformer_block(x1, pos1, gp1, idx1, n1, 256, p['t_d0'])

    x2, pos2 = _transition_down(x1, pos1, n1, n2, p['td1'])
    pos2_t = jnp.transpose(pos2, (0, 2, 1))
    pos2_pad = jnp.pad(pos2.reshape(2 * n2, 3), ((0, 0), (0, 125)))
    nbr2 = _knn_call(pos2, pos2_t, K, True)
    idx2 = _kmajor_idx(nbr2, n2)
    gp2 = _sc_gather(pos2_pad, idx2).reshape(2, K, n2, 128)
    x2 = _transformer_block(x2, pos2, gp2, idx2, n2, 512, p['t_d1'])

    x2 = _dense_relu_call(x2, p['summit']['W'], p['summit']['b'])
    x2 = _transformer_block(x2, pos2, gp2, idx2, n2, 512, p['t_sum'])

    xu1 = _transition_up(x1, x2, pos1, pos2, n1, n2,
                         p['tu1_sub'], p['tu1_mlp'])
    xu1 = _transformer_block(xu1, pos1, gp1, idx1, n1, 256, p['t_u1'])

    xu0 = _transition_up(x0, xu1, pos0, pos1, n0, n1,
                         p['tu0_sub'], p['tu0_mlp'])
    xu0 = _transformer_block(xu0, pos0, gp0, idx0, n0, 128, p['t_u0'])

    out = _head_call(xu0, p['head'])
    return out.reshape(2, n0, 3)
